# Initial kernel scaffold; baseline (speedup 1.0000x reference)
import jax
import jax.numpy as jnp
from jax.experimental import pallas as pl


def kernel(x, pair_features, pair_index, W_atom, b_atom, g_atom, be_atom, W_pair, b_pair, g_pair, be_pair, W_a2p, b_a2p):
    raise NotImplementedError("write your pallas kernel here")


# speedup vs baseline: 4.8339x; 4.8339x over previous
import functools

import jax
import jax.numpy as jnp
from jax import lax
from jax.experimental import pallas as pl
from jax.experimental.pallas import tpu as pltpu
from jax.experimental.pallas import tpu_sc as plsc

N = 10000
E = 320000
A_IN = 128
P_IN = 16
A_OUT = 128
P_OUT = 16
EPS = 1e-5

SC_CORES = 2
SC_SUBCORES = 16
NW = SC_CORES * SC_SUBCORES
EPW = E // NW
CHUNK = 2000
NCHUNK = EPW // CHUNK

PACK = 128 // P_OUT
EP = E // PACK
BE = 4000
NB = EP // BE


def _atom_body(x_ref, wat_ref, ba_ref, g_ref, be_ref, wst_ref, wrt_ref,
               atom_ref, xs_ref, xr_ref):
    x = x_ref[...]
    lin = jnp.dot(x, wat_ref[...], preferred_element_type=jnp.float32)
    h = jnp.maximum(lin + ba_ref[...], 0.0)
    m = jnp.mean(h, axis=0, keepdims=True)
    var = jnp.mean(h * h, axis=0, keepdims=True) - m * m
    inv = g_ref[...] * lax.rsqrt(var + EPS)
    atom_ref[...] = (h - m) * inv + be_ref[...]
    xs_ref[...] = jnp.dot(x, wst_ref[...], preferred_element_type=jnp.float32)
    xr_ref[...] = jnp.dot(x, wrt_ref[...], preferred_element_type=jnp.float32)


def _atom_call(x, wat, ba, g, be, wst, wrt):
    return pl.pallas_call(
        _atom_body,
        out_shape=[
            jax.ShapeDtypeStruct((N, A_OUT), jnp.float32),
            jax.ShapeDtypeStruct((N, P_OUT), jnp.float32),
            jax.ShapeDtypeStruct((N, P_OUT), jnp.float32),
        ],
    )(x, wat, ba, g, be, wst, wrt)


def _gather_body(xs_hbm, xr_hbm, idxs_hbm, idxr_hbm, out_hbm,
                 idx_s, idx_r, rows_s, rows_r, sem):
    wid = lax.axis_index("s") * SC_CORES + lax.axis_index("c")
    base = wid * EPW

    def chunk_body(k, carry):
        off = base + k * CHUNK
        pltpu.sync_copy(idxs_hbm.at[pl.ds(off, CHUNK)], idx_s)
        pltpu.sync_copy(idxr_hbm.at[pl.ds(off, CHUNK)], idx_r)
        cp1 = pltpu.async_copy(xs_hbm.at[idx_s], rows_s, sem)
        cp2 = pltpu.async_copy(xr_hbm.at[idx_r], rows_r, sem)
        cp1.wait()
        cp2.wait()

        def add_body(e, c):
            rows_s[e, :] = rows_s[e, :] + rows_r[e, :]
            return c

        lax.fori_loop(0, CHUNK, add_body, 0, unroll=4)
        pltpu.sync_copy(rows_s, out_hbm.at[pl.ds(off, CHUNK)])
        return carry

    lax.fori_loop(0, NCHUNK, chunk_body, 0)


def _gather_call(xs, xr, idx_src, idx_dst):
    run = pl.kernel(
        _gather_body,
        out_type=jax.ShapeDtypeStruct((E, P_OUT), jnp.float32),
        mesh=plsc.VectorSubcoreMesh(
            core_axis_name="c", subcore_axis_name="s",
            num_cores=SC_CORES, num_subcores=SC_SUBCORES),
        scratch_types=[
            pltpu.VMEM((CHUNK,), jnp.int32),
            pltpu.VMEM((CHUNK,), jnp.int32),
            pltpu.VMEM((CHUNK, P_OUT), jnp.float32),
            pltpu.VMEM((CHUNK, P_OUT), jnp.float32),
            pltpu.SemaphoreType.DMA,
        ],
        compiler_params=pltpu.CompilerParams(use_tc_tiling_on_sc=False),
    )
    return run(xs, xr, idx_src, idx_dst)


def _pair_body(pf_ref, gath_ref, wbdt_ref, bp_ref, g_ref, cb_ref,
               out_ref, acc_ref):
    p = pl.program_id(0)

    @pl.when((p == 0) & (pl.program_id(1) == 0))
    def _init():
        acc_ref[...] = jnp.zeros_like(acc_ref)

    lin = jnp.dot(pf_ref[...], wbdt_ref[...],
                  preferred_element_type=jnp.float32)
    h = jnp.maximum(lin + bp_ref[...], 0.0)

    @pl.when(p == 0)
    def _accumulate():
        acc_ref[0:1, :] += jnp.sum(h, axis=0, keepdims=True)
        acc_ref[1:2, :] += jnp.sum(h * h, axis=0, keepdims=True)

    @pl.when(p == 1)
    def _emit():
        r0 = lax.broadcasted_iota(jnp.int32, (128, 128), 0) % P_OUT
        r1 = lax.broadcasted_iota(jnp.int32, (128, 128), 1) % P_OUT
        fold = (r0 == r1).astype(jnp.float32)
        s1 = jnp.dot(acc_ref[0:1, :], fold, preferred_element_type=jnp.float32)
        s2 = jnp.dot(acc_ref[1:2, :], fold, preferred_element_type=jnp.float32)
        m = s1 * (1.0 / E)
        var = s2 * (1.0 / E) - m * m
        inv = g_ref[...] * lax.rsqrt(var + EPS)
        out_ref[...] = gath_ref[...] + cb_ref[...] + (h - m) * inv


def _pair_call(pf_p, gath_p, wbdt, bp_t, g_t, cb_t):
    return pl.pallas_call(
        _pair_body,
        grid=(2, NB),
        in_specs=[
            pl.BlockSpec((BE, 128), lambda p, j: (j, 0)),
            pl.BlockSpec((BE, 128), lambda p, j: (p * j, 0)),
            pl.BlockSpec((128, 128), lambda p, j: (0, 0)),
            pl.BlockSpec((1, 128), lambda p, j: (0, 0)),
            pl.BlockSpec((1, 128), lambda p, j: (0, 0)),
            pl.BlockSpec((1, 128), lambda p, j: (0, 0)),
        ],
        out_specs=pl.BlockSpec((BE, 128), lambda p, j: (p * j, 0)),
        out_shape=jax.ShapeDtypeStruct((EP, 128), jnp.float32),
        scratch_shapes=[pltpu.VMEM((2, 128), jnp.float32)],
    )(pf_p, gath_p, wbdt, bp_t, g_t, cb_t)


def kernel(x, pair_features, pair_index, W_atom, b_atom, g_atom, be_atom,
           W_pair, b_pair, g_pair, be_pair, W_a2p, b_a2p):
    f32 = jnp.float32
    wat = W_atom.T.astype(f32)
    wst = W_a2p[:, :A_IN].T.astype(f32)
    wrt = W_a2p[:, A_IN:].T.astype(f32)
    wbdt = jnp.kron(jnp.eye(PACK, dtype=f32), W_pair.T.astype(f32))
    bp_t = jnp.tile(b_pair, PACK).reshape(1, 128)
    g_t = jnp.tile(g_pair, PACK).reshape(1, 128)
    cb_t = jnp.tile(b_a2p + be_pair, PACK).reshape(1, 128)

    atom_out, xs, xr = _atom_call(
        x, wat, b_atom.reshape(1, A_OUT), g_atom.reshape(1, A_OUT),
        be_atom.reshape(1, A_OUT), wst, wrt)

    gath = _gather_call(xs, xr, pair_index[0], pair_index[1])

    pair_update = _pair_call(
        pair_features.reshape(EP, 128), gath.reshape(EP, 128),
        wbdt, bp_t, g_t, cb_t)

    return atom_out, pair_update.reshape(E, P_OUT)


# speedup vs baseline: 4.8726x; 1.0080x over previous
import jax
import jax.numpy as jnp
from jax import lax
from jax.experimental import pallas as pl
from jax.experimental.pallas import tpu as pltpu
from jax.experimental.pallas import tpu_sc as plsc

N = 10000
E = 320000
A_IN = 128
P_IN = 16
A_OUT = 128
P_OUT = 16
EPS = 1e-5

SC_CORES = 2
SC_SUBCORES = 16
NW = SC_CORES * SC_SUBCORES
EPW = E // NW
CHUNK = 1000
NCHUNK = EPW // CHUNK

PACK = 128 // P_OUT
EP = E // PACK
RPC = CHUNK // PACK
BE = 4000
NB = EP // BE


def _proj_body(x_ref, wst_ref, wrt_ref, xs_ref, xr_ref):
    x = x_ref[...]
    xs_ref[...] = jnp.dot(x, wst_ref[...], preferred_element_type=jnp.float32)
    xr_ref[...] = jnp.dot(x, wrt_ref[...], preferred_element_type=jnp.float32)


def _proj_call(x, wst, wrt):
    return pl.pallas_call(
        _proj_body,
        out_shape=[
            jax.ShapeDtypeStruct((N, P_OUT), jnp.float32),
            jax.ShapeDtypeStruct((N, P_OUT), jnp.float32),
        ],
    )(x, wst, wrt)


def _atom_body(x_ref, wat_ref, ba_ref, g_ref, be_ref, atom_ref):
    x = x_ref[...]
    lin = jnp.dot(x, wat_ref[...], preferred_element_type=jnp.float32)
    h = jnp.maximum(lin + ba_ref[...], 0.0)
    m = jnp.mean(h, axis=0, keepdims=True)
    var = jnp.mean(h * h, axis=0, keepdims=True) - m * m
    inv = g_ref[...] * lax.rsqrt(var + EPS)
    atom_ref[...] = (h - m) * inv + be_ref[...]


def _atom_call(x, wat, ba, g, be):
    return pl.pallas_call(
        _atom_body,
        out_shape=jax.ShapeDtypeStruct((N, A_OUT), jnp.float32),
    )(x, wat, ba, g, be)


def _gather_body(xs_hbm, xr_hbm, idxs_hbm, idxr_hbm, out_hbm,
                 idx_s, idx_r, rows_s0, rows_r0, rows_s1, rows_r1,
                 packed, sem0, sem1):
    wid = lax.axis_index("s") * SC_CORES + lax.axis_index("c")
    base = wid * EPW

    pltpu.sync_copy(idxs_hbm.at[pl.ds(base, EPW)], idx_s)
    pltpu.sync_copy(idxr_hbm.at[pl.ds(base, EPW)], idx_r)

    rows = ((rows_s0, rows_r0, sem0), (rows_s1, rows_r1, sem1))

    def issue(k):
        rs, rr, sem = rows[k % 2]
        sl = pl.ds(k * CHUNK, CHUNK)
        cs = pltpu.async_copy(xs_hbm.at[idx_s.at[sl]], rs, sem)
        cr = pltpu.async_copy(xr_hbm.at[idx_r.at[sl]], rr, sem)
        return cs, cr

    inflight = issue(0)
    for k in range(NCHUNK):
        nxt = issue(k + 1) if k + 1 < NCHUNK else None
        inflight[0].wait()
        inflight[1].wait()
        rs, rr, _ = rows[k % 2]

        def add_body(r, carry, rs=rs, rr=rr):
            for c in range(PACK):
                e = r * PACK + c
                packed[r, pl.ds(c * P_OUT, P_OUT)] = rs[e, :] + rr[e, :]
            return carry

        lax.fori_loop(0, RPC, add_body, 0)
        pltpu.sync_copy(packed, out_hbm.at[pl.ds(wid * (EPW // PACK) + k * RPC, RPC)])
        inflight = nxt


def _gather_call(xs, xr, idx_src, idx_dst):
    run = pl.kernel(
        _gather_body,
        out_type=jax.ShapeDtypeStruct((EP, 128), jnp.float32),
        mesh=plsc.VectorSubcoreMesh(
            core_axis_name="c", subcore_axis_name="s",
            num_cores=SC_CORES, num_subcores=SC_SUBCORES),
        scratch_types=[
            pltpu.VMEM((EPW,), jnp.int32),
            pltpu.VMEM((EPW,), jnp.int32),
            pltpu.VMEM((CHUNK, P_OUT), jnp.float32),
            pltpu.VMEM((CHUNK, P_OUT), jnp.float32),
            pltpu.VMEM((CHUNK, P_OUT), jnp.float32),
            pltpu.VMEM((CHUNK, P_OUT), jnp.float32),
            pltpu.VMEM((RPC, 128), jnp.float32),
            pltpu.SemaphoreType.DMA,
            pltpu.SemaphoreType.DMA,
        ],
        compiler_params=pltpu.CompilerParams(use_tc_tiling_on_sc=False),
    )
    return run(xs, xr, idx_src, idx_dst)


def _stats_body(pf_ref, wbdt_ref, bp_ref, out_ref):
    j = pl.program_id(0)

    @pl.when(j == 0)
    def _init():
        out_ref[...] = jnp.zeros_like(out_ref)

    lin = jnp.dot(pf_ref[...], wbdt_ref[...],
                  preferred_element_type=jnp.float32)
    h = jnp.maximum(lin + bp_ref[...], 0.0)
    out_ref[0:1, :] += jnp.sum(h, axis=0, keepdims=True)
    out_ref[1:2, :] += jnp.sum(h * h, axis=0, keepdims=True)


def _stats_call(pf_p, wbdt, bp_t):
    return pl.pallas_call(
        _stats_body,
        grid=(NB,),
        in_specs=[
            pl.BlockSpec((BE, 128), lambda j: (j, 0)),
            pl.BlockSpec((128, 128), lambda j: (0, 0)),
            pl.BlockSpec((1, 128), lambda j: (0, 0)),
        ],
        out_specs=pl.BlockSpec((8, 128), lambda j: (0, 0)),
        out_shape=jax.ShapeDtypeStruct((8, 128), jnp.float32),
    )(pf_p, wbdt, bp_t)


def _emit_body(pf_ref, gath_ref, stats_ref, wbdt_ref, bp_ref, g_ref, cb_ref,
               out_ref):
    lin = jnp.dot(pf_ref[...], wbdt_ref[...],
                  preferred_element_type=jnp.float32)
    h = jnp.maximum(lin + bp_ref[...], 0.0)
    r0 = lax.broadcasted_iota(jnp.int32, (128, 128), 0) % P_OUT
    r1 = lax.broadcasted_iota(jnp.int32, (128, 128), 1) % P_OUT
    fold = (r0 == r1).astype(jnp.float32)
    s1 = jnp.dot(stats_ref[0:1, :], fold, preferred_element_type=jnp.float32)
    s2 = jnp.dot(stats_ref[1:2, :], fold, preferred_element_type=jnp.float32)
    m = s1 * (1.0 / E)
    var = s2 * (1.0 / E) - m * m
    inv = g_ref[...] * lax.rsqrt(var + EPS)
    out_ref[...] = gath_ref[...] + cb_ref[...] + (h - m) * inv


def _emit_call(pf_p, gath_p, stats, wbdt, bp_t, g_t, cb_t):
    return pl.pallas_call(
        _emit_body,
        grid=(NB,),
        in_specs=[
            pl.BlockSpec((BE, 128), lambda j: (j, 0)),
            pl.BlockSpec((BE, 128), lambda j: (j, 0)),
            pl.BlockSpec((8, 128), lambda j: (0, 0)),
            pl.BlockSpec((128, 128), lambda j: (0, 0)),
            pl.BlockSpec((1, 128), lambda j: (0, 0)),
            pl.BlockSpec((1, 128), lambda j: (0, 0)),
            pl.BlockSpec((1, 128), lambda j: (0, 0)),
        ],
        out_specs=pl.BlockSpec((BE, 128), lambda j: (j, 0)),
        out_shape=jax.ShapeDtypeStruct((EP, 128), jnp.float32),
    )(pf_p, gath_p, stats, wbdt, bp_t, g_t, cb_t)


def kernel(x, pair_features, pair_index, W_atom, b_atom, g_atom, be_atom,
           W_pair, b_pair, g_pair, be_pair, W_a2p, b_a2p):
    f32 = jnp.float32
    wat = W_atom.T.astype(f32)
    wst = W_a2p[:, :A_IN].T.astype(f32)
    wrt = W_a2p[:, A_IN:].T.astype(f32)
    wbdt = jnp.kron(jnp.eye(PACK, dtype=f32), W_pair.T.astype(f32))
    bp_t = jnp.tile(b_pair, PACK).reshape(1, 128)
    g_t = jnp.tile(g_pair, PACK).reshape(1, 128)
    cb_t = jnp.tile(b_a2p + be_pair, PACK).reshape(1, 128)
    pf_p = pair_features.reshape(EP, 128)

    xs, xr = _proj_call(x, wst, wrt)
    gath_p = _gather_call(xs, xr, pair_index[0], pair_index[1])
    atom_out = _atom_call(
        x, wat, b_atom.reshape(1, A_OUT), g_atom.reshape(1, A_OUT),
        be_atom.reshape(1, A_OUT))
    stats = _stats_call(pf_p, wbdt, bp_t)
    pair_update = _emit_call(pf_p, gath_p, stats, wbdt, bp_t, g_t, cb_t)

    return atom_out, pair_update.reshape(E, P_OUT)


# speedup vs baseline: 11.1740x; 2.2932x over previous
import jax
import jax.numpy as jnp
from jax import lax
from jax.experimental import pallas as pl
from jax.experimental.pallas import tpu as pltpu
from jax.experimental.pallas import tpu_sc as plsc

N = 10000
E = 320000
A_IN = 128
P_IN = 16
A_OUT = 128
P_OUT = 16
EPS = 1e-5

SC_CORES = 2
SC_SUBCORES = 16
NW = SC_CORES * SC_SUBCORES

EPW = E // NW
CHUNK = 1000
NCHUNK = EPW // CHUNK

BEL = 64000
NB = E // BEL
QBEL = BEL // 4
GR = E // 4


def _proj_body(x_ref, wst_ref, wrt_ref, xs_ref, xr_ref):
    x = x_ref[...]
    xs_ref[...] = jnp.dot(x, wst_ref[...], preferred_element_type=jnp.float32)
    xr_ref[...] = jnp.dot(x, wrt_ref[...], preferred_element_type=jnp.float32)


def _proj_call(x, wst, wrt):
    return pl.pallas_call(
        _proj_body,
        out_shape=[
            jax.ShapeDtypeStruct((N, P_OUT), jnp.float32),
            jax.ShapeDtypeStruct((N, P_OUT), jnp.float32),
        ],
    )(x, wst, wrt)


def _atom_body(x_ref, wat_ref, ba_ref, g_ref, be_ref, atom_ref):
    x = x_ref[...]
    lin = jnp.dot(x, wat_ref[...], preferred_element_type=jnp.float32)
    h = jnp.maximum(lin + ba_ref[...], 0.0)
    m = jnp.mean(h, axis=0, keepdims=True)
    var = jnp.mean(h * h, axis=0, keepdims=True) - m * m
    inv = g_ref[...] * lax.rsqrt(var + EPS)
    atom_ref[...] = (h - m) * inv + be_ref[...]


def _atom_call(x, wat, ba, g, be):
    return pl.pallas_call(
        _atom_body,
        out_shape=jax.ShapeDtypeStruct((N, A_OUT), jnp.float32),
    )(x, wat, ba, g, be)


def _gather_body(xs_hbm, xr_hbm, idxs_hbm, idxr_hbm, out_hbm,
                 idx_s, idx_r, rows_s0, rows_r0, rows_s1, rows_r1,
                 sem0, sem1, semw0, semw1):
    wid = lax.axis_index("s") * SC_CORES + lax.axis_index("c")
    base = wid * EPW

    pltpu.sync_copy(idxs_hbm.at[pl.ds(base, EPW)], idx_s)
    pltpu.sync_copy(idxr_hbm.at[pl.ds(base, EPW)], idx_r)

    rows = ((rows_s0, rows_r0, sem0, semw0), (rows_s1, rows_r1, sem1, semw1))
    pending_w = [None, None]

    def issue(k):
        if pending_w[k % 2] is not None:
            pending_w[k % 2].wait()
            pending_w[k % 2] = None
        rs, rr, sem, _ = rows[k % 2]
        sl = pl.ds(k * CHUNK, CHUNK)
        cs = pltpu.async_copy(xs_hbm.at[idx_s.at[sl]], rs, sem)
        cr = pltpu.async_copy(xr_hbm.at[idx_r.at[sl]], rr, sem)
        return cs, cr

    inflight = issue(0)
    for k in range(NCHUNK):
        nxt = issue(k + 1) if k + 1 < NCHUNK else None
        inflight[0].wait()
        inflight[1].wait()
        rs, rr, _, semw = rows[k % 2]

        def add_body(e, carry, rs=rs, rr=rr):
            rs[e, :] = rs[e, :] + rr[e, :]
            return carry

        lax.fori_loop(0, CHUNK, add_body, 0, unroll=4)
        e0 = base + k * CHUNK
        row0 = (e0 // BEL) * QBEL + (e0 % QBEL)
        lane0 = ((e0 % BEL) // QBEL) * 32
        pending_w[k % 2] = pltpu.async_copy(
            rs, out_hbm.at[pl.ds(row0, CHUNK), pl.ds(lane0, P_OUT)], semw)
        inflight = nxt

    for w in pending_w:
        if w is not None:
            w.wait()


def _gather_call(xs, xr, idx_src, idx_dst):
    run = pl.kernel(
        _gather_body,
        out_type=jax.ShapeDtypeStruct((GR, 128), jnp.float32),
        mesh=plsc.VectorSubcoreMesh(
            core_axis_name="c", subcore_axis_name="s",
            num_cores=SC_CORES, num_subcores=SC_SUBCORES),
        scratch_types=[
            pltpu.VMEM((EPW,), jnp.int32),
            pltpu.VMEM((EPW,), jnp.int32),
            pltpu.VMEM((CHUNK, P_OUT), jnp.float32),
            pltpu.VMEM((CHUNK, P_OUT), jnp.float32),
            pltpu.VMEM((CHUNK, P_OUT), jnp.float32),
            pltpu.VMEM((CHUNK, P_OUT), jnp.float32),
            pltpu.SemaphoreType.DMA,
            pltpu.SemaphoreType.DMA,
            pltpu.SemaphoreType.DMA,
            pltpu.SemaphoreType.DMA,
        ],
        compiler_params=pltpu.CompilerParams(use_tc_tiling_on_sc=False),
    )
    return run(xs, xr, idx_src, idx_dst)


def _stats_body(pft_ref, wp_ref, bp_ref, s1_ref, s2_ref):
    j = pl.program_id(0)

    @pl.when(j == 0)
    def _init():
        s1_ref[...] = jnp.zeros_like(s1_ref)
        s2_ref[...] = jnp.zeros_like(s2_ref)

    lin = jnp.dot(wp_ref[...], pft_ref[...],
                  preferred_element_type=jnp.float32)
    h = jnp.maximum(lin + bp_ref[:, 0:1], 0.0)
    s1_ref[:, 0:1] += jnp.sum(h, axis=1, keepdims=True)
    s2_ref[:, 0:1] += jnp.sum(h * h, axis=1, keepdims=True)


def _stats_call(pf_t, wp, bp_c):
    return pl.pallas_call(
        _stats_body,
        grid=(NB,),
        in_specs=[
            pl.BlockSpec((P_OUT, BEL), lambda j: (0, j)),
            pl.BlockSpec((P_OUT, P_IN), lambda j: (0, 0)),
            pl.BlockSpec((P_OUT, 128), lambda j: (0, 0)),
        ],
        out_specs=[
            pl.BlockSpec((P_OUT, 128), lambda j: (0, 0)),
            pl.BlockSpec((P_OUT, 128), lambda j: (0, 0)),
        ],
        out_shape=[
            jax.ShapeDtypeStruct((P_OUT, 128), jnp.float32),
            jax.ShapeDtypeStruct((P_OUT, 128), jnp.float32),
        ],
    )(pf_t, wp, bp_c)


def _emit_body(pft_ref, gs_ref, s1_ref, s2_ref, wp_ref, bp_ref, g_ref,
               cb_ref, out_ref):
    lin = jnp.dot(wp_ref[...], pft_ref[...],
                  preferred_element_type=jnp.float32)
    h = jnp.maximum(lin + bp_ref[:, 0:1], 0.0)
    m = s1_ref[:, 0:1] * (1.0 / E)
    var = s2_ref[:, 0:1] * (1.0 / E) - m * m
    inv = g_ref[:, 0:1] * lax.rsqrt(var + EPS)
    r0 = lax.broadcasted_iota(jnp.int32, (P_OUT, P_OUT), 0)
    r1 = lax.broadcasted_iota(jnp.int32, (P_OUT, P_OUT), 1)
    eye = (r0 == r1).astype(jnp.float32)
    gts = [
        lax.dot_general(eye, gs_ref[:, pl.ds(32 * q, P_OUT)],
                        (((1,), (1,)), ((), ())),
                        preferred_element_type=jnp.float32)
        for q in range(4)
    ]
    gt = jnp.concatenate(gts, axis=1)
    out_ref[...] = gt + cb_ref[:, 0:1] + (h - m) * inv


def _emit_call(pf_t, gath, s1, s2, wp, bp_c, g_c, cb_c):
    return pl.pallas_call(
        _emit_body,
        grid=(NB,),
        in_specs=[
            pl.BlockSpec((P_OUT, BEL), lambda j: (0, j)),
            pl.BlockSpec((QBEL, 128), lambda j: (j, 0)),
            pl.BlockSpec((P_OUT, 128), lambda j: (0, 0)),
            pl.BlockSpec((P_OUT, 128), lambda j: (0, 0)),
            pl.BlockSpec((P_OUT, P_IN), lambda j: (0, 0)),
            pl.BlockSpec((P_OUT, 128), lambda j: (0, 0)),
            pl.BlockSpec((P_OUT, 128), lambda j: (0, 0)),
            pl.BlockSpec((P_OUT, 128), lambda j: (0, 0)),
        ],
        out_specs=pl.BlockSpec((P_OUT, BEL), lambda j: (0, j)),
        out_shape=jax.ShapeDtypeStruct((P_OUT, E), jnp.float32),
    )(pf_t, gath, s1, s2, wp, bp_c, g_c, cb_c)


def kernel(x, pair_features, pair_index, W_atom, b_atom, g_atom, be_atom,
           W_pair, b_pair, g_pair, be_pair, W_a2p, b_a2p):
    f32 = jnp.float32
    wat = W_atom.T.astype(f32)
    wst = W_a2p[:, :A_IN].T.astype(f32)
    wrt = W_a2p[:, A_IN:].T.astype(f32)
    bp_c = jnp.tile(b_pair.reshape(P_OUT, 1), (1, 128))
    g_c = jnp.tile(g_pair.reshape(P_OUT, 1), (1, 128))
    cb_c = jnp.tile((b_a2p + be_pair).reshape(P_OUT, 1), (1, 128))
    pf_t = pair_features.T

    xs, xr = _proj_call(x, wst, wrt)
    gath = _gather_call(xs, xr, pair_index[0], pair_index[1])
    atom_out = _atom_call(
        x, wat, b_atom.reshape(1, A_OUT), g_atom.reshape(1, A_OUT),
        be_atom.reshape(1, A_OUT))
    s1, s2 = _stats_call(pf_t, W_pair, bp_c)
    out_t = _emit_call(pf_t, gath, s1, s2, W_pair, bp_c, g_c, cb_c)

    return atom_out, out_t.T
